# baseline (speedup 1.0000x reference)
import functools

import jax
import jax.numpy as jnp
from jax import lax
from jax.experimental import pallas as pl
from jax.experimental.pallas import tpu as pltpu
from jax.experimental.pallas import tpu_sc as plsc

N = 50000
D = 64
E = 800000
B = 16384
L = 3
ALPHA = 1.0 / (L + 1)

NC, NS = 2, 16
HD = D // 2
NP = 50176
EP = 802816
SB = 256
EIT = EP // NS // (2 * SB)
HALF = EIT // 2
RCH = 112
NCHUNK = NP // NS // RCH
BS = B // NS

MLP_BLK = 2048
MLP_GRID = B // MLP_BLK


def _rsqrt16(x):
  i = lax.bitcast_convert_type(x, jnp.int32)
  i = jnp.int32(0x5F3759DF) - lax.shift_right_logical(i, jnp.int32(1))
  y = lax.bitcast_convert_type(i, jnp.float32)
  for _ in range(3):
    y = y * (1.5 - 0.5 * x * y * y)
  return jnp.where(x > 0.5, y, 0.0)


def _sc_lightgcn(cidx_hbm, elir_hbm, emb2_hbm,
                 h4_hbm, g_hbm, out_hbm, dis_hbm,
                 a_buf, o_buf, rows_a, rows_b, cix, ixp,
                 acc, sem_ixa, sem_ixb, sem_ga, sem_gb, sem_sa, sem_sb):
  c = lax.axis_index("c")
  s = lax.axis_index("s")
  cnp = c * NP

  zero16 = jnp.zeros((16,), jnp.float32)
  one16 = jnp.ones((16,), jnp.float32)

  def fill_rows(buf, val):
    def fr(i, _):
      buf[i, pl.ds(0, 16)] = val
      buf[i, pl.ds(16, 16)] = val
      return 0
    lax.fori_loop(0, SB, fr, 0)

  def offs(blk):
    for j in range(2):
      for m in range(SB // 16):
        ixp[blk, j, pl.ds(m * 16, 16)] = (
            cix[blk, 0, j, pl.ds(m * 16, 16)] + cnp)

  fill_rows(rows_a, zero16)

  def zero_acc(k, _):
    pltpu.sync_copy(rows_a.at[pl.ds(0, RCH)],
                    acc.at[pl.ds((s * NCHUNK + k) * RCH, RCH)])
    return 0
  lax.fori_loop(0, NCHUNK, zero_acc, 0)
  fill_rows(rows_a, one16)
  fill_rows(rows_b, one16)
  plsc.subcore_barrier()

  def deg_it(h, _):
    b0 = s * EIT + 2 * h
    pltpu.sync_copy(cidx_hbm.at[pl.ds(b0, 2)], cix)
    s0 = pltpu.async_copy(rows_a, acc.at[cix.at[0, 1, 0]], sem_sa, add=True)
    s1 = pltpu.async_copy(rows_b, acc.at[cix.at[0, 1, 1]], sem_sb, add=True)
    s2 = pltpu.async_copy(rows_a, acc.at[cix.at[1, 1, 0]], sem_ga, add=True)
    s3 = pltpu.async_copy(rows_b, acc.at[cix.at[1, 1, 1]], sem_gb, add=True)
    s0.wait()
    s1.wait()
    s2.wait()
    s3.wait()
    return 0
  lax.fori_loop(0, HALF, deg_it, 0)
  plsc.subcore_barrier()

  fill_rows(rows_a, zero16)

  def pass_a(k, _):
    r0 = (s * NCHUNK + k) * RCH
    pltpu.sync_copy(acc.at[pl.ds(r0, RCH)], a_buf)
    pltpu.sync_copy(rows_a.at[pl.ds(0, RCH)], acc.at[pl.ds(r0, RCH)])
    pltpu.sync_copy(emb2_hbm.at[pl.ds(cnp + r0, RCH)], o_buf)

    def vdis(i, _):
      a_buf[i, pl.ds(0, 16)] = _rsqrt16(a_buf[i, pl.ds(0, 16)])
      a_buf[i, pl.ds(16, 16)] = _rsqrt16(a_buf[i, pl.ds(16, 16)])
      return 0
    lax.fori_loop(0, RCH, vdis, 0)
    pltpu.sync_copy(a_buf, dis_hbm.at[pl.ds(cnp + r0, RCH)])

    def vg(i, _):
      a_buf[i, pl.ds(0, 16)] = a_buf[i, pl.ds(0, 16)] * o_buf[i, pl.ds(0, 16)]
      a_buf[i, pl.ds(16, 16)] = a_buf[i, pl.ds(16, 16)] * o_buf[i, pl.ds(16, 16)]
      return 0
    lax.fori_loop(0, RCH, vg, 0)
    pltpu.sync_copy(a_buf, g_hbm.at[pl.ds(cnp + r0, RCH)])
    pltpu.sync_copy(o_buf, out_hbm.at[pl.ds(cnp + r0, RCH)])
    return 0
  lax.fori_loop(0, NCHUNK, pass_a, 0)
  plsc.subcore_barrier()

  for l in range(L):
    last = l == L - 1

    pltpu.async_copy(cidx_hbm.at[s * EIT], cix.at[0], sem_ixa)

    def edge_it(h, _):
      b0 = s * EIT + 2 * h
      pltpu.make_async_copy(cidx_hbm.at[b0], cix.at[0], sem_ixa).wait()
      offs(0)
      hixb = pltpu.async_copy(cidx_hbm.at[b0 + 1], cix.at[1], sem_ixb)
      ga = pltpu.async_copy(g_hbm.at[ixp.at[0, 0]], rows_a, sem_ga)
      gb = pltpu.async_copy(g_hbm.at[ixp.at[0, 1]], rows_b, sem_gb)
      hixb.wait()
      offs(1)
      ga.wait()
      ga2 = pltpu.async_copy(g_hbm.at[ixp.at[1, 0]], rows_a, sem_ga)
      gb.wait()
      gb2 = pltpu.async_copy(g_hbm.at[ixp.at[1, 1]], rows_b, sem_gb)

      @pl.when(h < HALF - 1)
      def _():
        pltpu.async_copy(cidx_hbm.at[b0 + 2], cix.at[0], sem_ixa)

      ga2.wait()
      gb2.wait()
      return 0
    lax.fori_loop(0, HALF, edge_it, 0)
    plsc.subcore_barrier()

    fill_rows(rows_a, zero16)

    def node_pass(k, _):
      r0 = (s * NCHUNK + k) * RCH
      pltpu.sync_copy(acc.at[pl.ds(r0, RCH)], a_buf)
      if not last:
        pltpu.sync_copy(rows_a.at[pl.ds(0, RCH)], acc.at[pl.ds(r0, RCH)])
      pltpu.sync_copy(dis_hbm.at[pl.ds(cnp + r0, RCH)], o_buf)

      def vx(i, _):
        for j in (0, 16):
          x = a_buf[i, pl.ds(j, 16)] * o_buf[i, pl.ds(j, 16)]
          a_buf[i, pl.ds(j, 16)] = x
          o_buf[i, pl.ds(j, 16)] = x * o_buf[i, pl.ds(j, 16)]
        return 0
      lax.fori_loop(0, RCH, vx, 0)
      if not last:
        pltpu.sync_copy(o_buf, g_hbm.at[pl.ds(cnp + r0, RCH)])
      pltpu.sync_copy(out_hbm.at[pl.ds(cnp + r0, RCH)], o_buf)

      def vo(i, _):
        for j in (0, 16):
          o_buf[i, pl.ds(j, 16)] = o_buf[i, pl.ds(j, 16)] + a_buf[i, pl.ds(j, 16)]
        return 0
      lax.fori_loop(0, RCH, vo, 0)
      pltpu.sync_copy(o_buf, out_hbm.at[pl.ds(cnp + r0, RCH)])
      return 0
    lax.fori_loop(0, NCHUNK, node_pass, 0)
    plsc.subcore_barrier()

  for p in range(2):
    def h4_it(k, _):
      it0 = p * (B // (2 * SB)) + s * (BS // (2 * SB)) + k
      pltpu.sync_copy(elir_hbm.at[it0], cix.at[0, 0])
      offs(0)
      ga = pltpu.async_copy(out_hbm.at[ixp.at[0, 0]], rows_a, sem_ga)
      gb = pltpu.async_copy(out_hbm.at[ixp.at[0, 1]], rows_b, sem_gb)
      row0 = (2 * p + c) * B + s * BS + k * 2 * SB
      ga.wait()
      pltpu.sync_copy(rows_a, h4_hbm.at[pl.ds(row0, SB)])
      gb.wait()
      pltpu.sync_copy(rows_b, h4_hbm.at[pl.ds(row0 + SB, SB)])
      return 0
    lax.fori_loop(0, BS // (2 * SB), h4_it, 0)


_sc_call = functools.partial(
    pl.kernel,
    out_type=[
        jax.ShapeDtypeStruct((4 * B, HD), jnp.float32),
        jax.ShapeDtypeStruct((2 * NP, HD), jnp.float32),
        jax.ShapeDtypeStruct((2 * NP, HD), jnp.float32),
        jax.ShapeDtypeStruct((2 * NP, HD), jnp.float32),
    ],
    mesh=plsc.VectorSubcoreMesh(core_axis_name="c", subcore_axis_name="s"),
    compiler_params=pltpu.CompilerParams(use_tc_tiling_on_sc=False),
    scratch_types=[
        pltpu.VMEM((RCH, HD), jnp.float32),
        pltpu.VMEM((RCH, HD), jnp.float32),
        pltpu.VMEM((SB, HD), jnp.float32),
        pltpu.VMEM((SB, HD), jnp.float32),
        pltpu.VMEM((2, 2, 2, SB), jnp.int32),
        pltpu.VMEM((2, 2, SB), jnp.int32),
        pltpu.VMEM_SHARED((NP, HD), jnp.float32),
        pltpu.SemaphoreType.DMA,
        pltpu.SemaphoreType.DMA,
        pltpu.SemaphoreType.DMA,
        pltpu.SemaphoreType.DMA,
        pltpu.SemaphoreType.DMA,
        pltpu.SemaphoreType.DMA,
    ],
)(_sc_lightgcn)


def _mlp_body(h4_ref, w1_ref, b1_ref, w2_ref, b2_ref, y_ref, pred_ref, ls_ref):
  i = pl.program_id(0)
  h = jnp.concatenate(
      [h4_ref[0], h4_ref[1], h4_ref[2], h4_ref[3]], axis=1)
  z = jnp.dot(h, w1_ref[...], preferred_element_type=jnp.float32)
  hr = jnp.maximum(z * ALPHA + b1_ref[...], 0.0)
  p = jnp.sum(hr * w2_ref[...], axis=1, keepdims=True) + b2_ref[0, 0]
  pred_ref[...] = p
  d = p - y_ref[...]
  part = jnp.sum(d * d)

  @pl.when(i == 0)
  def _():
    ls_ref[...] = part.reshape(1, 1)

  @pl.when(i > 0)
  def _():
    ls_ref[...] = ls_ref[...] + part.reshape(1, 1)

  @pl.when(i == MLP_GRID - 1)
  def _():
    ls_ref[...] = ls_ref[...] * (1.0 / B)


_mlp_call = pl.pallas_call(
    _mlp_body,
    grid=(MLP_GRID,),
    in_specs=[
        pl.BlockSpec((4, MLP_BLK, HD), lambda i: (0, i, 0)),
        pl.BlockSpec((2 * D, D), lambda i: (0, 0)),
        pl.BlockSpec((1, D), lambda i: (0, 0)),
        pl.BlockSpec((1, D), lambda i: (0, 0)),
        pl.BlockSpec((1, 1), lambda i: (0, 0)),
        pl.BlockSpec((MLP_BLK, 1), lambda i: (i, 0)),
    ],
    out_specs=[
        pl.BlockSpec((MLP_BLK, 1), lambda i: (i, 0)),
        pl.BlockSpec((1, 1), lambda i: (0, 0)),
    ],
    out_shape=[
        jax.ShapeDtypeStruct((B, 1), jnp.float32),
        jax.ShapeDtypeStruct((1, 1), jnp.float32),
    ],
)


def kernel(edge_index, edge_label_index, edge_label, emb, W1, b1, W2, b2):
  srcr = jnp.pad(edge_index[0], (0, EP - E),
                 constant_values=N).reshape(EP // (2 * SB), 2, SB)
  dstr = jnp.pad(edge_index[1], (0, EP - E),
                 constant_values=N).reshape(EP // (2 * SB), 2, SB)
  cidx = jnp.stack([srcr, dstr], axis=1)
  elir = edge_label_index.reshape(2 * B // (2 * SB), 2, SB)
  emb2 = (jnp.pad(emb, ((0, NP - N), (0, 0)))
          .reshape(NP, 2, HD).transpose(1, 0, 2).reshape(2 * NP, HD))

  h4, _, _, _ = _sc_call(cidx, elir, emb2)
  h4 = h4.reshape(4, B, HD)

  pred, ls = _mlp_call(h4, W1, b1.reshape(1, D), W2.reshape(1, D),
                       b2.reshape(1, 1), edge_label.reshape(B, 1))
  return pred, ls[0, 0]


# speedup vs baseline: 1.0029x; 1.0029x over previous
import functools

import jax
import jax.numpy as jnp
from jax import lax
from jax.experimental import pallas as pl
from jax.experimental.pallas import tpu as pltpu
from jax.experimental.pallas import tpu_sc as plsc

N = 50000
D = 64
E = 800000
B = 16384
L = 3
ALPHA = 1.0 / (L + 1)

NC, NS = 2, 16
HD = D // 2
NP = 50176
EP = 802816
QR = 128
EIT = EP // NS // 512
HALF = EIT // 2
RCH = 56
NCHUNK = NP // NS // RCH
BS = B // NS

MLP_BLK = 2048
MLP_GRID = B // MLP_BLK


def _rsqrt16(x):
  i = lax.bitcast_convert_type(x, jnp.int32)
  i = jnp.int32(0x5F3759DF) - lax.shift_right_logical(i, jnp.int32(1))
  y = lax.bitcast_convert_type(i, jnp.float32)
  for _ in range(3):
    y = y * (1.5 - 0.5 * x * y * y)
  return jnp.where(x > 0.5, y, 0.0)


def _sc_lightgcn(cidx_hbm, elir_hbm, emb2_hbm,
                 h4_hbm, g_hbm, out_hbm, dis_hbm,
                 ax0, dx0, ox0, ax1, dx1, ox1,
                 r0, r1, r2, r3, cix, ixp, acc,
                 sem_ixa, sem_ixb,
                 sem_g0, sem_g1, sem_g2, sem_g3,
                 sem_s0, sem_s1, sem_s2, sem_s3):
  c = lax.axis_index("c")
  s = lax.axis_index("s")
  cnp = c * NP
  rows = (r0, r1, r2, r3)
  sg = (sem_g0, sem_g1, sem_g2, sem_g3)
  ss = (sem_s0, sem_s1, sem_s2, sem_s3)

  zero16 = jnp.zeros((16,), jnp.float32)
  one16 = jnp.ones((16,), jnp.float32)

  def fill_rows(buf, val):
    def fr(i, _):
      buf[i, pl.ds(0, 16)] = val
      buf[i, pl.ds(16, 16)] = val
      return 0
    lax.fori_loop(0, QR, fr, 0)

  def offs(blk):
    for q in range(4):
      for m in range(QR // 16):
        ixp[blk, q, pl.ds(m * 16, 16)] = (
            cix[blk, 0, q, pl.ds(m * 16, 16)] + cnp)

  fill_rows(r0, zero16)

  def zero_acc(k, _):
    pltpu.sync_copy(r0.at[pl.ds(0, RCH)],
                    acc.at[pl.ds((s * NCHUNK + k) * RCH, RCH)])
    return 0
  lax.fori_loop(0, NCHUNK, zero_acc, 0)
  for r in rows:
    fill_rows(r, one16)
  plsc.subcore_barrier()

  pltpu.async_copy(cidx_hbm.at[s * EIT], cix.at[0], sem_ixa)

  def deg_it(h, _):
    b0 = s * EIT + 2 * h
    pltpu.make_async_copy(cidx_hbm.at[b0], cix.at[0], sem_ixa).wait()
    hixb = pltpu.async_copy(cidx_hbm.at[b0 + 1], cix.at[1], sem_ixb)
    sa = [pltpu.async_copy(rows[q], acc.at[cix.at[0, 1, q]], ss[q], add=True)
          for q in range(4)]
    hixb.wait()
    sb = []
    for q in range(4):
      sa[q].wait()
      sb.append(pltpu.async_copy(rows[q], acc.at[cix.at[1, 1, q]],
                                 sg[q], add=True))

    @pl.when(h < HALF - 1)
    def _():
      pltpu.async_copy(cidx_hbm.at[b0 + 2], cix.at[0], sem_ixa)

    for h_ in sb:
      h_.wait()
    return 0
  lax.fori_loop(0, HALF, deg_it, 0)
  plsc.subcore_barrier()

  fill_rows(r0, zero16)

  def dense_pass(reads, compute, writes):
    def body(kk, _):
      ra = reads(0, 2 * kk)
      rb = reads(1, 2 * kk + 1)
      for h_ in ra:
        h_.wait()
      compute(0)
      wa = writes(0, 2 * kk)
      for h_ in rb:
        h_.wait()
      compute(1)
      wb = writes(1, 2 * kk + 1)
      for h_ in wa + wb:
        h_.wait()
      return 0
    lax.fori_loop(0, NCHUNK // 2, body, 0)

  sets = ((ax0, dx0, ox0), (ax1, dx1, ox1))

  def chunk_base(k):
    return (s * NCHUNK + k) * RCH

  rsem = ((sem_g0, sem_g1, sem_g2), (sem_g3, sem_s1, sem_s2))
  wsem = ((sem_g0, sem_g1, sem_g2, sem_s0), (sem_g3, sem_s1, sem_s2, sem_s3))

  def pa_reads(t, k):
    ax, dx, ox = sets[t]
    rr = chunk_base(k)
    return [
        pltpu.async_copy(acc.at[pl.ds(rr, RCH)], ax, rsem[t][0]),
        pltpu.async_copy(emb2_hbm.at[pl.ds(cnp + rr, RCH)], ox, rsem[t][1]),
    ]

  def pa_compute(t):
    ax, dx, ox = sets[t]

    def v(i, _):
      for j in (0, 16):
        d = _rsqrt16(ax[i, pl.ds(j, 16)])
        ax[i, pl.ds(j, 16)] = d
        dx[i, pl.ds(j, 16)] = d * ox[i, pl.ds(j, 16)]
      return 0
    lax.fori_loop(0, RCH, v, 0)

  def pa_writes(t, k):
    ax, dx, ox = sets[t]
    rr = chunk_base(k)
    return [
        pltpu.async_copy(r0.at[pl.ds(0, RCH)], acc.at[pl.ds(rr, RCH)],
                         wsem[t][0]),
        pltpu.async_copy(ax, dis_hbm.at[pl.ds(cnp + rr, RCH)], wsem[t][1]),
        pltpu.async_copy(dx, g_hbm.at[pl.ds(cnp + rr, RCH)], wsem[t][2]),
        pltpu.async_copy(ox, out_hbm.at[pl.ds(cnp + rr, RCH)], wsem[t][3]),
    ]

  dense_pass(pa_reads, pa_compute, pa_writes)
  plsc.subcore_barrier()

  for l in range(L):
    last = l == L - 1

    pltpu.async_copy(cidx_hbm.at[s * EIT], cix.at[0], sem_ixa)

    def edge_it(h, _):
      b0 = s * EIT + 2 * h
      pltpu.make_async_copy(cidx_hbm.at[b0], cix.at[0], sem_ixa).wait()
      offs(0)
      hixb = pltpu.async_copy(cidx_hbm.at[b0 + 1], cix.at[1], sem_ixb)

      def gat(blk, q):
        return pltpu.async_copy(g_hbm.at[ixp.at[blk, q]], rows[q], sg[q])

      def scat(blk, q):
        return pltpu.async_copy(rows[q], acc.at[cix.at[blk, 1, q]],
                                ss[q], add=True)

      g = [None] * 8
      sc = [None] * 8
      g[0] = gat(0, 0)
      g[1] = gat(0, 1)
      g[0].wait(); sc[0] = scat(0, 0); g[2] = gat(0, 2)
      g[1].wait(); sc[1] = scat(0, 1); g[3] = gat(0, 3)
      hixb.wait()
      offs(1)
      g[2].wait(); sc[0].wait(); sc[2] = scat(0, 2); g[4] = gat(1, 0)
      g[3].wait(); sc[1].wait(); sc[3] = scat(0, 3); g[5] = gat(1, 1)
      g[4].wait(); sc[2].wait(); sc[4] = scat(1, 0); g[6] = gat(1, 2)
      g[5].wait(); sc[3].wait(); sc[5] = scat(1, 1); g[7] = gat(1, 3)

      @pl.when(h < HALF - 1)
      def _():
        pltpu.async_copy(cidx_hbm.at[b0 + 2], cix.at[0], sem_ixa)

      g[6].wait(); sc[4].wait(); sc[6] = scat(1, 2)
      g[7].wait(); sc[5].wait(); sc[7] = scat(1, 3)
      sc[6].wait()
      sc[7].wait()
      return 0
    lax.fori_loop(0, HALF, edge_it, 0)
    plsc.subcore_barrier()

    fill_rows(r0, zero16)

    def np_reads(t, k):
      ax, dx, ox = sets[t]
      rr = chunk_base(k)
      return [
          pltpu.async_copy(acc.at[pl.ds(rr, RCH)], ax, rsem[t][0]),
          pltpu.async_copy(dis_hbm.at[pl.ds(cnp + rr, RCH)], dx, rsem[t][1]),
          pltpu.async_copy(out_hbm.at[pl.ds(cnp + rr, RCH)], ox, rsem[t][2]),
      ]

    def np_compute(t):
      ax, dx, ox = sets[t]

      def v(i, _):
        for j in (0, 16):
          x = ax[i, pl.ds(j, 16)] * dx[i, pl.ds(j, 16)]
          dx[i, pl.ds(j, 16)] = x * dx[i, pl.ds(j, 16)]
          ox[i, pl.ds(j, 16)] = ox[i, pl.ds(j, 16)] + x
        return 0
      lax.fori_loop(0, RCH, v, 0)

    def np_writes(t, k, last=last):
      ax, dx, ox = sets[t]
      rr = chunk_base(k)
      w = [pltpu.async_copy(ox, out_hbm.at[pl.ds(cnp + rr, RCH)],
                            wsem[t][0])]
      if not last:
        w.append(pltpu.async_copy(r0.at[pl.ds(0, RCH)],
                                  acc.at[pl.ds(rr, RCH)], wsem[t][1]))
        w.append(pltpu.async_copy(dx, g_hbm.at[pl.ds(cnp + rr, RCH)],
                                  wsem[t][2]))
      return w

    dense_pass(np_reads, np_compute, np_writes)
    plsc.subcore_barrier()

  for p in range(2):
    def h4_it(k, _):
      it0 = p * (B // 512) + s * (BS // 512) + k
      pltpu.sync_copy(elir_hbm.at[it0], cix.at[0, 0])
      offs(0)
      row0 = (2 * p + c) * B + s * BS + k * 512

      def gat(q):
        return pltpu.async_copy(out_hbm.at[ixp.at[0, q]], rows[q], sg[q])

      gh = [gat(0), gat(1)]
      for q in range(4):
        gh[q].wait()
        if q + 2 < 4:
          gh.append(gat(q + 2))
        pltpu.sync_copy(rows[q], h4_hbm.at[pl.ds(row0 + q * QR, QR)])
      return 0
    lax.fori_loop(0, BS // 512, h4_it, 0)


_sc_call = functools.partial(
    pl.kernel,
    out_type=[
        jax.ShapeDtypeStruct((4 * B, HD), jnp.float32),
        jax.ShapeDtypeStruct((2 * NP, HD), jnp.float32),
        jax.ShapeDtypeStruct((2 * NP, HD), jnp.float32),
        jax.ShapeDtypeStruct((2 * NP, HD), jnp.float32),
    ],
    mesh=plsc.VectorSubcoreMesh(core_axis_name="c", subcore_axis_name="s"),
    compiler_params=pltpu.CompilerParams(use_tc_tiling_on_sc=False),
    scratch_types=[
        pltpu.VMEM((RCH, HD), jnp.float32),
        pltpu.VMEM((RCH, HD), jnp.float32),
        pltpu.VMEM((RCH, HD), jnp.float32),
        pltpu.VMEM((RCH, HD), jnp.float32),
        pltpu.VMEM((RCH, HD), jnp.float32),
        pltpu.VMEM((RCH, HD), jnp.float32),
        pltpu.VMEM((QR, HD), jnp.float32),
        pltpu.VMEM((QR, HD), jnp.float32),
        pltpu.VMEM((QR, HD), jnp.float32),
        pltpu.VMEM((QR, HD), jnp.float32),
        pltpu.VMEM((2, 2, 4, QR), jnp.int32),
        pltpu.VMEM((2, 4, QR), jnp.int32),
        pltpu.VMEM_SHARED((NP, HD), jnp.float32),
        pltpu.SemaphoreType.DMA,
        pltpu.SemaphoreType.DMA,
        pltpu.SemaphoreType.DMA,
        pltpu.SemaphoreType.DMA,
        pltpu.SemaphoreType.DMA,
        pltpu.SemaphoreType.DMA,
        pltpu.SemaphoreType.DMA,
        pltpu.SemaphoreType.DMA,
        pltpu.SemaphoreType.DMA,
        pltpu.SemaphoreType.DMA,
    ],
)(_sc_lightgcn)


def _mlp_body(h4_ref, w1_ref, b1_ref, w2_ref, b2_ref, y_ref, pred_ref, ls_ref):
  i = pl.program_id(0)
  h = jnp.concatenate(
      [h4_ref[0], h4_ref[1], h4_ref[2], h4_ref[3]], axis=1)
  z = jnp.dot(h, w1_ref[...], preferred_element_type=jnp.float32)
  hr = jnp.maximum(z * ALPHA + b1_ref[...], 0.0)
  p = jnp.sum(hr * w2_ref[...], axis=1, keepdims=True) + b2_ref[0, 0]
  pred_ref[...] = p
  d = p - y_ref[...]
  part = jnp.sum(d * d)

  @pl.when(i == 0)
  def _():
    ls_ref[...] = part.reshape(1, 1)

  @pl.when(i > 0)
  def _():
    ls_ref[...] = ls_ref[...] + part.reshape(1, 1)

  @pl.when(i == MLP_GRID - 1)
  def _():
    ls_ref[...] = ls_ref[...] * (1.0 / B)


_mlp_call = pl.pallas_call(
    _mlp_body,
    grid=(MLP_GRID,),
    in_specs=[
        pl.BlockSpec((4, MLP_BLK, HD), lambda i: (0, i, 0)),
        pl.BlockSpec((2 * D, D), lambda i: (0, 0)),
        pl.BlockSpec((1, D), lambda i: (0, 0)),
        pl.BlockSpec((1, D), lambda i: (0, 0)),
        pl.BlockSpec((1, 1), lambda i: (0, 0)),
        pl.BlockSpec((MLP_BLK, 1), lambda i: (i, 0)),
    ],
    out_specs=[
        pl.BlockSpec((MLP_BLK, 1), lambda i: (i, 0)),
        pl.BlockSpec((1, 1), lambda i: (0, 0)),
    ],
    out_shape=[
        jax.ShapeDtypeStruct((B, 1), jnp.float32),
        jax.ShapeDtypeStruct((1, 1), jnp.float32),
    ],
)


def kernel(edge_index, edge_label_index, edge_label, emb, W1, b1, W2, b2):
  srcr = jnp.pad(edge_index[0], (0, EP - E),
                 constant_values=N).reshape(EP // 512, 4, QR)
  dstr = jnp.pad(edge_index[1], (0, EP - E),
                 constant_values=N).reshape(EP // 512, 4, QR)
  cidx = jnp.stack([srcr, dstr], axis=1)
  elir = edge_label_index.reshape(2 * B // 512, 4, QR)
  emb2 = (jnp.pad(emb, ((0, NP - N), (0, 0)))
          .reshape(NP, 2, HD).transpose(1, 0, 2).reshape(2 * NP, HD))

  h4, _, _, _ = _sc_call(cidx, elir, emb2)
  h4 = h4.reshape(4, B, HD)

  pred, ls = _mlp_call(h4, W1, b1.reshape(1, D), W2.reshape(1, D),
                       b2.reshape(1, 1), edge_label.reshape(B, 1))
  return pred, ls[0, 0]
